# baseline (speedup 1.0000x reference)
import functools

import jax
import jax.numpy as jnp
from jax import lax
from jax.experimental import pallas as pl
from jax.experimental.pallas import tpu as pltpu
from jax.experimental.pallas import tpu_sc as plsc

_BATCH = 16384
_ROWS = 1000
_D_IN = 64
_D_HID = 128
_D_OUT = 64
_BLK = 2048


def _make_sc_gather():
    info = plsc.get_sparse_core_info()
    nw = info.num_cores * info.num_subcores
    bpw = _BATCH // nw
    ch = 128
    nch = bpw // ch
    mesh = plsc.VectorSubcoreMesh(core_axis_name="c", subcore_axis_name="s")

    @functools.partial(
        pl.kernel,
        mesh=mesh,
        out_type=jax.ShapeDtypeStruct((_BATCH, 2 * _D_IN), jnp.float32),
        scratch_types=[
            pltpu.VMEM((bpw,), jnp.int32),
            pltpu.VMEM((bpw, 2 * _D_IN), jnp.float32),
            pltpu.SemaphoreType.DMA,
        ],
    )
    def gather(tbl_hbm, idx_hbm, out_hbm, idx_v, rows_v, sem):
        wid = lax.axis_index("s") * info.num_cores + lax.axis_index("c")
        base = wid * bpw
        pltpu.sync_copy(idx_hbm.at[pl.ds(base, bpw)], idx_v)
        copies = [
            pltpu.async_copy(
                tbl_hbm.at[idx_v.at[pl.ds(j * ch, ch)]],
                rows_v.at[pl.ds(j * ch, ch)],
                sem,
            )
            for j in range(nch)
        ]
        for c in copies:
            c.wait()
        pltpu.sync_copy(rows_v, out_hbm.at[pl.ds(base, bpw)])

    return gather


_sc_gather = _make_sc_gather()


def _mlp_body(x_ref, w1_ref, b1_ref, w2_ref, b2_ref, out_ref):
    x = x_ref[:, :_D_IN]
    h = jnp.dot(x, w1_ref[...], preferred_element_type=jnp.float32) + b1_ref[...]
    h = h * jnp.tanh(jax.nn.softplus(h))
    out_ref[...] = (
        jnp.dot(h, w2_ref[...], preferred_element_type=jnp.float32) + b2_ref[...]
    )


def _batch_mlp(x2, W1, b1, W2, b2):
    n_blk = _BATCH // _BLK
    return pl.pallas_call(
        _mlp_body,
        grid=(n_blk,),
        in_specs=[
            pl.BlockSpec((_BLK, 2 * _D_IN), lambda i: (i, 0)),
            pl.BlockSpec((_D_IN, _D_HID), lambda i: (0, 0)),
            pl.BlockSpec((1, _D_HID), lambda i: (0, 0)),
            pl.BlockSpec((_D_HID, _D_OUT), lambda i: (0, 0)),
            pl.BlockSpec((1, _D_OUT), lambda i: (0, 0)),
        ],
        out_specs=pl.BlockSpec((_BLK, _D_OUT), lambda i: (i, 0)),
        out_shape=jax.ShapeDtypeStruct((_BATCH, _D_OUT), jnp.float32),
    )(x2, W1, b1.reshape(1, -1), W2, b2.reshape(1, -1))


def kernel(diffusion_step, embedding, W1, b1, W2, b2):
    emb128 = jnp.pad(embedding, ((0, 0), (0, _D_IN)))
    idx = diffusion_step.astype(jnp.int32)
    x2 = _sc_gather(emb128, idx)
    return _batch_mlp(x2, W1, b1, W2, b2)


# speedup vs baseline: 1.1966x; 1.1966x over previous
import functools

import jax
import jax.numpy as jnp
from jax import lax
from jax.experimental import pallas as pl
from jax.experimental.pallas import tpu as pltpu
from jax.experimental.pallas import tpu_sc as plsc

_BATCH = 16384
_ROWS = 1000
_D_IN = 64
_D_HID = 128
_D_OUT = 64
_BLK = 2048


def _make_sc_gather():
    info = plsc.get_sparse_core_info()
    nw = info.num_cores * info.num_subcores
    bpw = _BATCH // nw
    ch = 128
    nch = bpw // ch
    mesh = plsc.VectorSubcoreMesh(core_axis_name="c", subcore_axis_name="s")

    @functools.partial(
        pl.kernel,
        mesh=mesh,
        out_type=jax.ShapeDtypeStruct((_BATCH, 2 * _D_IN), jnp.float32),
        scratch_types=[
            pltpu.VMEM((bpw,), jnp.int32),
            pltpu.VMEM((bpw, 2 * _D_IN), jnp.float32),
            pltpu.SemaphoreType.DMA,
        ],
    )
    def gather(tbl_hbm, idx_hbm, out_hbm, idx_v, rows_v, sem):
        wid = lax.axis_index("s") * info.num_cores + lax.axis_index("c")
        base = wid * bpw
        pltpu.sync_copy(idx_hbm.at[pl.ds(base, bpw)], idx_v)
        copies = [
            pltpu.async_copy(
                tbl_hbm.at[idx_v.at[pl.ds(j * ch, ch)]],
                rows_v.at[pl.ds(j * ch, ch)],
                sem,
            )
            for j in range(nch)
        ]
        for c in copies:
            c.wait()
        pltpu.sync_copy(rows_v, out_hbm.at[pl.ds(base, bpw)])

    return gather


_sc_gather = _make_sc_gather()


def _mlp_body(x_ref, w1_ref, b1_ref, w2t_ref, b2_ref, out_ref):
    x = x_ref[:, :_D_IN]
    h = jnp.dot(x, w1_ref[...], preferred_element_type=jnp.float32) + b1_ref[...]
    h = h * jnp.tanh(jax.nn.softplus(h))
    out_ref[...] = (
        jax.lax.dot_general(
            w2t_ref[...], h, (((1,), (1,)), ((), ())),
            preferred_element_type=jnp.float32,
        )
        + b2_ref[...]
    )


def _batch_mlp_t(x2, W1, b1, W2t, b2):
    n_blk = _BATCH // _BLK
    return pl.pallas_call(
        _mlp_body,
        grid=(n_blk,),
        in_specs=[
            pl.BlockSpec((_BLK, 2 * _D_IN), lambda i: (i, 0)),
            pl.BlockSpec((_D_IN, _D_HID), lambda i: (0, 0)),
            pl.BlockSpec((1, _D_HID), lambda i: (0, 0)),
            pl.BlockSpec((_D_OUT, _D_HID), lambda i: (0, 0)),
            pl.BlockSpec((_D_OUT, 1), lambda i: (0, 0)),
        ],
        out_specs=pl.BlockSpec((_D_OUT, _BLK), lambda i: (0, i)),
        out_shape=jax.ShapeDtypeStruct((_D_OUT, _BATCH), jnp.float32),
    )(x2, W1, b1.reshape(1, -1), W2t, b2.reshape(-1, 1))


def kernel(diffusion_step, embedding, W1, b1, W2, b2):
    emb128 = jnp.pad(embedding, ((0, 0), (0, _D_IN)))
    idx = diffusion_step.astype(jnp.int32)
    x2 = _sc_gather(emb128, idx)
    out_t = _batch_mlp_t(x2, W1, b1, W2.T, b2)
    return out_t.T


# speedup vs baseline: 1.2346x; 1.0318x over previous
import functools

import jax
import jax.numpy as jnp
from jax import lax
from jax.experimental import pallas as pl
from jax.experimental.pallas import tpu as pltpu
from jax.experimental.pallas import tpu_sc as plsc

_BATCH = 16384
_ROWS = 1000
_D_IN = 64
_D_HID = 128
_D_OUT = 64
_BLK = 2048


def _make_sc_gather():
    info = plsc.get_sparse_core_info()
    nw = info.num_cores * info.num_subcores
    bpw = _BATCH // nw
    ch = 128
    nch = bpw // ch
    mesh = plsc.VectorSubcoreMesh(core_axis_name="c", subcore_axis_name="s")

    @functools.partial(
        pl.kernel,
        mesh=mesh,
        out_type=jax.ShapeDtypeStruct((_BATCH, 2 * _D_IN), jnp.float32),
        scratch_types=[
            pltpu.VMEM((bpw,), jnp.int32),
            pltpu.VMEM((bpw, 2 * _D_IN), jnp.float32),
            pltpu.SemaphoreType.DMA,
        ],
    )
    def gather(tbl_hbm, idx_hbm, out_hbm, idx_v, rows_v, sem):
        wid = lax.axis_index("s") * info.num_cores + lax.axis_index("c")
        base = wid * bpw
        pltpu.sync_copy(idx_hbm.at[pl.ds(base, bpw)], idx_v)
        copies = [
            pltpu.async_copy(
                tbl_hbm.at[idx_v.at[pl.ds(j * ch, ch)]],
                rows_v.at[pl.ds(j * ch, ch)],
                sem,
            )
            for j in range(nch)
        ]
        for c in copies:
            c.wait()
        pltpu.sync_copy(rows_v, out_hbm.at[pl.ds(base, bpw)])

    return gather


_sc_gather = _make_sc_gather()


def _mlp_body(x_ref, w1_ref, b1_ref, w2t_ref, b2_ref, out_ref):
    x = x_ref[:, :_D_IN]
    h = jnp.dot(x, w1_ref[...], preferred_element_type=jnp.float32) + b1_ref[...]
    u = jnp.exp(h)
    num = u * (u + 2.0)
    h = h * num / (num + 2.0)
    out_ref[...] = (
        jax.lax.dot_general(
            w2t_ref[...], h, (((1,), (1,)), ((), ())),
            preferred_element_type=jnp.float32,
        )
        + b2_ref[...]
    )


def _batch_mlp_t(x2, W1, b1, W2t, b2):
    n_blk = _BATCH // _BLK
    return pl.pallas_call(
        _mlp_body,
        grid=(n_blk,),
        in_specs=[
            pl.BlockSpec((_BLK, 2 * _D_IN), lambda i: (i, 0)),
            pl.BlockSpec((_D_IN, _D_HID), lambda i: (0, 0)),
            pl.BlockSpec((1, _D_HID), lambda i: (0, 0)),
            pl.BlockSpec((_D_OUT, _D_HID), lambda i: (0, 0)),
            pl.BlockSpec((_D_OUT, 1), lambda i: (0, 0)),
        ],
        out_specs=pl.BlockSpec((_D_OUT, _BLK), lambda i: (0, i)),
        out_shape=jax.ShapeDtypeStruct((_D_OUT, _BATCH), jnp.float32),
    )(x2, W1, b1.reshape(1, -1), W2t, b2.reshape(-1, 1))


def kernel(diffusion_step, embedding, W1, b1, W2, b2):
    emb128 = jnp.pad(embedding, ((0, 0), (0, _D_IN)))
    idx = diffusion_step.astype(jnp.int32)
    x2 = _sc_gather(emb128, idx)
    out_t = _batch_mlp_t(x2, W1, b1, W2.T, b2)
    return out_t.T


# speedup vs baseline: 1.2995x; 1.0525x over previous
import functools

import jax
import jax.numpy as jnp
from jax import lax
from jax.experimental import pallas as pl
from jax.experimental.pallas import tpu as pltpu
from jax.experimental.pallas import tpu_sc as plsc

_BATCH = 16384
_ROWS = 1000
_D_IN = 64
_D_HID = 128
_D_OUT = 64
_BLK = 4096


def _make_sc_gather():
    info = plsc.get_sparse_core_info()
    nw = info.num_cores * info.num_subcores
    bpw = _BATCH // nw
    ch = 128
    nch = bpw // ch
    mesh = plsc.VectorSubcoreMesh(core_axis_name="c", subcore_axis_name="s")

    @functools.partial(
        pl.kernel,
        mesh=mesh,
        out_type=jax.ShapeDtypeStruct((_BATCH, 2 * _D_IN), jnp.float32),
        scratch_types=[
            pltpu.VMEM((bpw,), jnp.int32),
            pltpu.VMEM((bpw, 2 * _D_IN), jnp.float32),
            pltpu.SemaphoreType.DMA,
        ],
    )
    def gather(tbl_hbm, idx_hbm, out_hbm, idx_v, rows_v, sem):
        wid = lax.axis_index("s") * info.num_cores + lax.axis_index("c")
        base = wid * bpw
        pltpu.sync_copy(idx_hbm.at[pl.ds(base, bpw)], idx_v)
        copies = [
            pltpu.async_copy(
                tbl_hbm.at[idx_v.at[pl.ds(j * ch, ch)]],
                rows_v.at[pl.ds(j * ch, ch)],
                sem,
            )
            for j in range(nch)
        ]
        for c in copies:
            c.wait()
        pltpu.sync_copy(rows_v, out_hbm.at[pl.ds(base, bpw)])

    return gather


_sc_gather = _make_sc_gather()


def _mlp_body(x_ref, w1_ref, b1_ref, w2t_ref, b2_ref, out_ref):
    x = x_ref[:, :_D_IN]
    h = jnp.dot(x, w1_ref[...], preferred_element_type=jnp.float32) + b1_ref[...]
    u = jnp.exp(h)
    num = u * (u + 2.0)
    h = h * num / (num + 2.0)
    out_ref[...] = (
        jax.lax.dot_general(
            w2t_ref[...], h, (((1,), (1,)), ((), ())),
            preferred_element_type=jnp.float32,
        )
        + b2_ref[...]
    )


def _batch_mlp_t(x2, W1, b1, W2t, b2):
    n_blk = _BATCH // _BLK
    return pl.pallas_call(
        _mlp_body,
        grid=(n_blk,),
        in_specs=[
            pl.BlockSpec((_BLK, 2 * _D_IN), lambda i: (i, 0)),
            pl.BlockSpec((_D_IN, _D_HID), lambda i: (0, 0)),
            pl.BlockSpec((1, _D_HID), lambda i: (0, 0)),
            pl.BlockSpec((_D_OUT, _D_HID), lambda i: (0, 0)),
            pl.BlockSpec((_D_OUT, 1), lambda i: (0, 0)),
        ],
        out_specs=pl.BlockSpec((_D_OUT, _BLK), lambda i: (0, i)),
        out_shape=jax.ShapeDtypeStruct((_D_OUT, _BATCH), jnp.float32),
    )(x2, W1, b1.reshape(1, -1), W2t, b2.reshape(-1, 1))


def kernel(diffusion_step, embedding, W1, b1, W2, b2):
    emb128 = jnp.pad(embedding, ((0, 0), (0, _D_IN)))
    idx = diffusion_step.astype(jnp.int32)
    x2 = _sc_gather(emb128, idx)
    out_t = _batch_mlp_t(x2, W1, b1, W2.T, b2)
    return out_t.T


# speedup vs baseline: 1.3416x; 1.0324x over previous
import functools

import jax
import jax.numpy as jnp
from jax import lax
from jax.experimental import pallas as pl
from jax.experimental.pallas import tpu as pltpu
from jax.experimental.pallas import tpu_sc as plsc

_BATCH = 16384
_ROWS = 1000
_D_IN = 64
_D_HID = 128
_D_OUT = 64
_BLK = 8192


def _make_sc_gather():
    info = plsc.get_sparse_core_info()
    nw = info.num_cores * info.num_subcores
    bpw = _BATCH // nw
    ch = 128
    nch = bpw // ch
    mesh = plsc.VectorSubcoreMesh(core_axis_name="c", subcore_axis_name="s")

    @functools.partial(
        pl.kernel,
        mesh=mesh,
        out_type=jax.ShapeDtypeStruct((_BATCH, 2 * _D_IN), jnp.float32),
        scratch_types=[
            pltpu.VMEM((bpw,), jnp.int32),
            pltpu.VMEM((bpw, 2 * _D_IN), jnp.float32),
            pltpu.SemaphoreType.DMA,
        ],
    )
    def gather(tbl_hbm, idx_hbm, out_hbm, idx_v, rows_v, sem):
        wid = lax.axis_index("s") * info.num_cores + lax.axis_index("c")
        base = wid * bpw
        pltpu.sync_copy(idx_hbm.at[pl.ds(base, bpw)], idx_v)
        copies = [
            pltpu.async_copy(
                tbl_hbm.at[idx_v.at[pl.ds(j * ch, ch)]],
                rows_v.at[pl.ds(j * ch, ch)],
                sem,
            )
            for j in range(nch)
        ]
        for c in copies:
            c.wait()
        pltpu.sync_copy(rows_v, out_hbm.at[pl.ds(base, bpw)])

    return gather


_sc_gather = _make_sc_gather()


def _mlp_body(x_ref, w1_ref, b1_ref, w2t_ref, b2_ref, out_ref):
    x = x_ref[:, :_D_IN]
    h = jnp.dot(x, w1_ref[...], preferred_element_type=jnp.float32) + b1_ref[...]
    u = jnp.exp(h)
    num = u * (u + 2.0)
    h = h * num / (num + 2.0)
    out_ref[...] = (
        jax.lax.dot_general(
            w2t_ref[...], h, (((1,), (1,)), ((), ())),
            preferred_element_type=jnp.float32,
        )
        + b2_ref[...]
    )


def _batch_mlp_t(x2, W1, b1, W2t, b2):
    n_blk = _BATCH // _BLK
    return pl.pallas_call(
        _mlp_body,
        grid=(n_blk,),
        in_specs=[
            pl.BlockSpec((_BLK, 2 * _D_IN), lambda i: (i, 0)),
            pl.BlockSpec((_D_IN, _D_HID), lambda i: (0, 0)),
            pl.BlockSpec((1, _D_HID), lambda i: (0, 0)),
            pl.BlockSpec((_D_OUT, _D_HID), lambda i: (0, 0)),
            pl.BlockSpec((_D_OUT, 1), lambda i: (0, 0)),
        ],
        out_specs=pl.BlockSpec((_D_OUT, _BLK), lambda i: (0, i)),
        out_shape=jax.ShapeDtypeStruct((_D_OUT, _BATCH), jnp.float32),
    )(x2, W1, b1.reshape(1, -1), W2t, b2.reshape(-1, 1))


def kernel(diffusion_step, embedding, W1, b1, W2, b2):
    emb128 = jnp.pad(embedding, ((0, 0), (0, _D_IN)))
    idx = diffusion_step.astype(jnp.int32)
    x2 = _sc_gather(emb128, idx)
    out_t = _batch_mlp_t(x2, W1, b1, W2.T, b2)
    return out_t.T
